# Initial kernel scaffold; baseline (speedup 1.0000x reference)
import functools

import jax
import jax.numpy as jnp
from jax import lax
from jax.experimental import pallas as pl
from jax.experimental.pallas import tpu as pltpu
from jax.experimental.pallas import tpu_sc as plsc

VOCAB = 1000000
HIDDEN = 64
BATCH = 16384
HIST = 50

NC = 2
NS = 16
NW = NC * NS

TOTAL = BATCH * HIST
PER_W = TOTAL // NW
CHUNK = 128
NSTEPS = PER_W // CHUNK


def _make_kernel():
  mesh = plsc.VectorSubcoreMesh(
      core_axis_name="c", subcore_axis_name="s", num_cores=NC, num_subcores=NS
  )

  @functools.partial(
      pl.kernel,
      out_type=jax.ShapeDtypeStruct((TOTAL, HIDDEN), jnp.float32),
      mesh=mesh,
      scratch_types=[
          pltpu.VMEM((NSTEPS, CHUNK), jnp.int32),
          pltpu.VMEM((CHUNK, HIDDEN), jnp.float32),
          pltpu.SemaphoreType.DMA,
      ],
  )
  def k(idx_hbm, table_hbm, out_hbm, idx_v, rows_v, sem):
    wid = lax.axis_index("s") * NC + lax.axis_index("c")
    base = wid * PER_W
    pltpu.sync_copy(idx_hbm.at[wid], idx_v)

    @pl.loop(0, NSTEPS)
    def _(j):
      pltpu.async_copy(table_hbm.at[idx_v.at[j]], rows_v, sem).wait()
      pltpu.sync_copy(rows_v, out_hbm.at[pl.ds(base + j * CHUNK, CHUNK)])

  return k


_kernel_call = _make_kernel()


@jax.jit
def kernel(v, weight):
  idx = v.reshape(NW, NSTEPS, CHUNK)
  out = _kernel_call(idx, weight)
  return out.reshape(BATCH, HIST, HIDDEN)


# speedup vs baseline: 1.6849x; 1.6849x over previous
import functools

import jax
import jax.numpy as jnp
from jax import lax
from jax.experimental import pallas as pl
from jax.experimental.pallas import tpu as pltpu
from jax.experimental.pallas import tpu_sc as plsc

VOCAB = 1000000
HIDDEN = 64
BATCH = 16384
HIST = 50

NC = 2
NS = 16
NW = NC * NS

TOTAL = BATCH * HIST
PER_W = TOTAL // NW
CHUNK = 128
NSTEPS = PER_W // CHUNK


def _make_kernel():
  mesh = plsc.VectorSubcoreMesh(
      core_axis_name="c", subcore_axis_name="s", num_cores=NC, num_subcores=NS
  )

  @functools.partial(
      pl.kernel,
      out_type=jax.ShapeDtypeStruct((TOTAL, HIDDEN), jnp.float32),
      mesh=mesh,
      scratch_types=[
          pltpu.VMEM((NSTEPS, CHUNK), jnp.int32),
          pltpu.VMEM((CHUNK, HIDDEN), jnp.float32),
          pltpu.SemaphoreType.DMA,
      ],
      compiler_params=pltpu.CompilerParams(use_tc_tiling_on_sc=False),
  )
  def k(idx_hbm, table_hbm, out_hbm, idx_v, rows_v, sem):
    wid = lax.axis_index("s") * NC + lax.axis_index("c")
    base = wid * PER_W
    pltpu.sync_copy(idx_hbm.at[wid], idx_v)

    @pl.loop(0, NSTEPS)
    def _(j):
      pltpu.async_copy(table_hbm.at[idx_v.at[j]], rows_v, sem).wait()
      pltpu.sync_copy(rows_v, out_hbm.at[pl.ds(base + j * CHUNK, CHUNK)])

  return k


_kernel_call = _make_kernel()


@jax.jit
def kernel(v, weight):
  idx = v.reshape(NW, NSTEPS, CHUNK)
  out = _kernel_call(idx, weight)
  return out.reshape(BATCH, HIST, HIDDEN)


# speedup vs baseline: 1.8717x; 1.1109x over previous
import functools

import jax
import jax.numpy as jnp
from jax import lax
from jax.experimental import pallas as pl
from jax.experimental.pallas import tpu as pltpu
from jax.experimental.pallas import tpu_sc as plsc

VOCAB = 1000000
HIDDEN = 64
BATCH = 16384
HIST = 50

NC = 2
NS = 16
NW = NC * NS

TOTAL = BATCH * HIST
PER_W = TOTAL // NW
CHUNK = 128
NSTEPS = PER_W // CHUNK


NBUF = 8


def _make_kernel():
  mesh = plsc.VectorSubcoreMesh(
      core_axis_name="c", subcore_axis_name="s", num_cores=NC, num_subcores=NS
  )

  @functools.partial(
      pl.kernel,
      out_type=jax.ShapeDtypeStruct((TOTAL, HIDDEN), jnp.float32),
      mesh=mesh,
      scratch_types=[
          pltpu.VMEM((NSTEPS, CHUNK), jnp.int32),
          [pltpu.VMEM((CHUNK, HIDDEN), jnp.float32) for _ in range(NBUF)],
          [pltpu.SemaphoreType.DMA for _ in range(NBUF)],
          [pltpu.SemaphoreType.DMA for _ in range(NBUF)],
      ],
      compiler_params=pltpu.CompilerParams(use_tc_tiling_on_sc=False),
  )
  def k(idx_hbm, table_hbm, out_hbm, idx_v, bufs, g_sems, w_sems):
    wid = lax.axis_index("s") * NC + lax.axis_index("c")
    base = wid * PER_W
    pltpu.sync_copy(idx_hbm.at[wid], idx_v)

    def gather(c, b):
      pltpu.async_copy(table_hbm.at[idx_v.at[c]], bufs[b], g_sems[b])

    def gather_wait(c, b):
      pltpu.make_async_copy(table_hbm.at[idx_v.at[c]], bufs[b], g_sems[b]).wait()

    def wb(c, b):
      pltpu.async_copy(bufs[b], out_hbm.at[pl.ds(base + c * CHUNK, CHUNK)], w_sems[b])

    def wb_wait(c, b):
      pltpu.make_async_copy(
          bufs[b], out_hbm.at[pl.ds(base + c * CHUNK, CHUNK)], w_sems[b]
      ).wait()

    for b in range(NBUF):
      gather(b, b)

    @pl.loop(NBUF, NSTEPS, step=NBUF)
    def _(c):
      for b in range(NBUF):
        gather_wait(c - NBUF + b, b)
        wb(c - NBUF + b, b)
      for b in range(NBUF):
        wb_wait(c - NBUF + b, b)
        gather(c + b, b)

    for b in range(NBUF):
      gather_wait(NSTEPS - NBUF + b, b)
      wb(NSTEPS - NBUF + b, b)
    for b in range(NBUF):
      wb_wait(NSTEPS - NBUF + b, b)

  return k


_kernel_call = _make_kernel()


@jax.jit
def kernel(v, weight):
  idx = v.reshape(NW, NSTEPS, CHUNK)
  out = _kernel_call(idx, weight)
  return out.reshape(BATCH, HIST, HIDDEN)
